# baseline (speedup 1.0000x reference)
import functools

import jax
import jax.numpy as jnp
from jax import lax
from jax.experimental import pallas as pl
from jax.experimental.pallas import tpu as pltpu
from jax.experimental.pallas import tpu_sc as plsc

_NC = 2
_NS = 16
_NW = _NC * _NS


@functools.partial(jax.jit, static_argnames=("chunk",))
def _emb_call(x_flat, w1, w2, *, chunk):
    bf = x_flat.shape[0]
    pivot = w1.shape[0]
    d = w1.shape[1]
    per_w = bf // _NW
    nchunk = per_w // chunk
    assert per_w % chunk == 0 and bf % _NW == 0 and nchunk >= 2

    mesh = plsc.VectorSubcoreMesh(
        core_axis_name="c", subcore_axis_name="s",
        num_cores=_NC, num_subcores=_NS,
    )

    @functools.partial(
        pl.kernel,
        out_type=jax.ShapeDtypeStruct((bf, d), jnp.float32),
        mesh=mesh,
        compiler_params=pltpu.CompilerParams(
            needs_layout_passes=False, use_tc_tiling_on_sc=False,
        ),
        scratch_types=[
            pltpu.VMEM((2, chunk), jnp.int32),
            pltpu.VMEM((2, chunk), jnp.int32),
            pltpu.VMEM((2, chunk), jnp.int32),
            pltpu.VMEM((2, chunk, d), jnp.float32),
            pltpu.VMEM((2, chunk, d), jnp.float32),
            pltpu.SemaphoreType.DMA,
            pltpu.SemaphoreType.DMA,
            pltpu.SemaphoreType.DMA,
            pltpu.SemaphoreType.DMA,
        ],
    )
    def emb(x_hbm, w1_hbm, w2_hbm, out_hbm, xv, i1v, i2v, r1v, r2v,
            sg0, sg1, so0, so1):
        wid = lax.axis_index("s") * _NC + lax.axis_index("c")
        base = wid * per_w
        iota = lax.iota(jnp.int32, 16)
        gsems = (sg0, sg1)
        osems = (so0, so1)

        def stage(k, s):
            cbase = base + k * chunk
            pltpu.sync_copy(x_hbm.at[pl.ds(cbase, chunk)], xv.at[s])

            def prep(g, c):
                xx = xv[s, pl.ds(g * 16, 16)]
                m = xx < pivot
                i1v[s, pl.ds(g * 16, 16)] = jnp.where(m, xx, 0)
                i2v[s, pl.ds(g * 16, 16)] = jnp.where(m, 0, xx - pivot)
                return c

            lax.fori_loop(0, chunk // 16, prep, 0, unroll=4)
            pltpu.async_copy(w1_hbm.at[i1v.at[s]], r1v.at[s], gsems[s])
            pltpu.async_copy(w2_hbm.at[i2v.at[s]], r2v.at[s], gsems[s])

        def wait_gathers(s):
            pltpu.make_async_copy(
                w1_hbm.at[i1v.at[s]], r1v.at[s], gsems[s]).wait()
            pltpu.make_async_copy(
                w2_hbm.at[i2v.at[s]], r2v.at[s], gsems[s]).wait()

        def start_out(k, s):
            cbase = base + k * chunk
            pltpu.async_copy(r1v.at[s], out_hbm.at[pl.ds(cbase, chunk)],
                             osems[s])

        def wait_out(k, s):
            cbase = base + k * chunk
            pltpu.make_async_copy(
                r1v.at[s], out_hbm.at[pl.ds(cbase, chunk)], osems[s]).wait()

        def merge(s):
            svec = jnp.full((16,), s, jnp.int32)

            def merge16(g, c):
                rbase = g * 16
                for r in range(16):
                    row = rbase + r
                    rowvec = jnp.full((16,), row, jnp.int32)
                    mval = plsc.load_gather(xv, [svec, rowvec])
                    m2 = mval >= pivot
                    for v in range(d // 16):
                        b = r2v[s, row, pl.ds(v * 16, 16)]
                        plsc.store_scatter(
                            r1v, [svec, rowvec, v * 16 + iota], b, mask=m2)
                return c

            lax.fori_loop(0, chunk // 16, merge16, 0)

        stage(0, 0)

        def chunk_body(j, c):
            is0 = lax.rem(j, 2) == 0

            @pl.when(jnp.logical_and(j + 1 < nchunk, is0))
            def _stage_to_slot1():
                @pl.when(j >= 1)
                def _():
                    wait_out(j - 1, 1)
                stage(j + 1, 1)

            @pl.when(jnp.logical_and(j + 1 < nchunk, jnp.logical_not(is0)))
            def _stage_to_slot0():
                wait_out(j - 1, 0)
                stage(j + 1, 0)

            @pl.when(is0)
            def _consume0():
                wait_gathers(0)
                merge(0)
                start_out(j, 0)

            @pl.when(jnp.logical_not(is0))
            def _consume1():
                wait_gathers(1)
                merge(1)
                start_out(j, 1)

            return c

        lax.fori_loop(0, nchunk, chunk_body, 0)

        wait_out(nchunk - 2, (nchunk - 2) % 2)
        wait_out(nchunk - 1, (nchunk - 1) % 2)

    return emb(x_flat, w1, w2)


def kernel(x, w1, w2):
    b, f = x.shape
    d = w1.shape[1]
    flat = x.reshape(-1).astype(jnp.int32)
    out = _emb_call(flat, w1, w2, chunk=256)
    return out.reshape(b, f, d)
